# baseline (speedup 1.0000x reference)
import jax
import jax.numpy as jnp
from jax import lax
from jax.experimental import pallas as pl
from jax.experimental.pallas import tpu as pltpu
from jax.experimental.pallas import tpu_sc as plsc

N = 10000
NP = 10240
E = 320000
EB = 128
NBLK = 2560
EP = NBLK * EB
NC = 2
NS = 16
NBW = NBLK // (NC * NS)
RPS = NP // NS
F = 128

_MESH = plsc.VectorSubcoreMesh(
    core_axis_name="c", subcore_axis_name="s", num_cores=NC, num_subcores=NS)


NBD = NBLK // NS


def _deg_body(src2d, dst2d, deg_src, deg_dst, idx_v, ones_v, zb_v, hist_sh):
    c = lax.axis_index("c")
    sid = lax.axis_index("s")

    one16 = jnp.full((16,), 1.0, jnp.float32)
    zero16 = jnp.zeros((16,), jnp.float32)
    for j in range(EB // 16):
        ones_v[pl.ds(j * 16, 16)] = one16

    @pl.loop(0, RPS // 16)
    def _(j):
        zb_v[pl.ds(j * 16, 16)] = zero16

    pltpu.sync_copy(zb_v, hist_sh.at[pl.ds(sid * RPS, RPS)])

    @pl.when(c == 0)
    def _():
        pltpu.sync_copy(src2d.at[pl.ds(sid * NBD, NBD)], idx_v)

    @pl.when(c == 1)
    def _():
        pltpu.sync_copy(dst2d.at[pl.ds(sid * NBD, NBD)], idx_v)

    plsc.subcore_barrier()

    @pl.loop(0, NBD)
    def _(b):
        pltpu.sync_copy(ones_v, hist_sh.at[idx_v.at[b]], add=True)

    plsc.subcore_barrier()

    pltpu.sync_copy(hist_sh.at[pl.ds(sid * RPS, RPS)], zb_v)

    @pl.when(c == 0)
    def _():
        pltpu.sync_copy(zb_v, deg_src.at[pl.ds(sid * RPS, RPS)])

    @pl.when(c == 1)
    def _():
        pltpu.sync_copy(zb_v, deg_dst.at[pl.ds(sid * RPS, RPS)])


_deg_call = pl.kernel(
    _deg_body,
    out_type=(jax.ShapeDtypeStruct((NP,), jnp.float32),
              jax.ShapeDtypeStruct((NP,), jnp.float32)),
    mesh=_MESH,
    scratch_types=[
        pltpu.VMEM((NBD, EB), jnp.int32),
        pltpu.VMEM((EB,), jnp.float32),
        pltpu.VMEM((RPS,), jnp.float32),
        pltpu.VMEM_SHARED((NP,), jnp.float32),
    ],
)


CHK = 40
NCH = NBW // CHK


def _edge_body(src2d, dst2d, h_hbm, out_hbm,
               srcb, dstb, rows0, rows1, sem0, sem1, agg_sh):
    c = lax.axis_index("c")
    sid = lax.axis_index("s")
    base = pl.multiple_of(c * NP, NP)
    bb = (c * NS + sid) * NBW

    z16 = jnp.zeros((16,), jnp.float32)

    @pl.loop(0, EB)
    def _(r):
        for j in range(F // 16):
            rows0[r, pl.ds(j * 16, 16)] = z16

    for k in range(RPS // EB):
        pltpu.sync_copy(rows0, agg_sh.at[pl.ds(sid * RPS + k * EB, EB)])
    plsc.subcore_barrier()

    def gstart(b, buf, sem):
        pltpu.async_copy(h_hbm.at[srcb.at[b]], buf, sem)

    def gwait(b, buf, sem):
        pltpu.make_async_copy(h_hbm.at[srcb.at[b]], buf, sem).wait()

    def scat(b, buf):
        pltpu.sync_copy(buf, agg_sh.at[dstb.at[b]], add=True)

    @pl.loop(0, NCH)
    def _(ch):
        pltpu.sync_copy(src2d.at[pl.ds(bb + ch * CHK, CHK)], srcb)
        pltpu.sync_copy(dst2d.at[pl.ds(bb + ch * CHK, CHK)], dstb)
        gstart(0, rows0, sem0)

        @pl.loop(0, CHK // 2)
        def _(i):
            b0 = i * 2
            gstart(b0 + 1, rows1, sem1)
            gwait(b0, rows0, sem0)
            scat(b0, rows0)
            b1 = b0 + 1

            @pl.when(b1 + 1 < CHK)
            def _():
                gstart(b1 + 1, rows0, sem0)

            gwait(b1, rows1, sem1)
            scat(b1, rows1)

    plsc.subcore_barrier()

    for k in range(RPS // EB):
        r = sid * RPS + k * EB
        pltpu.sync_copy(agg_sh.at[pl.ds(r, EB)], rows0)
        pltpu.sync_copy(rows0, out_hbm.at[pl.ds(base + r, EB)])


_edge_call = pl.kernel(
    _edge_body,
    out_type=jax.ShapeDtypeStruct((2 * NP, F), jnp.float32),
    mesh=_MESH,
    scratch_types=[
        pltpu.VMEM((CHK, EB), jnp.int32),
        pltpu.VMEM((CHK, EB), jnp.int32),
        pltpu.VMEM((EB, F), jnp.float32),
        pltpu.VMEM((EB, F), jnp.float32),
        pltpu.SemaphoreType.DMA,
        pltpu.SemaphoreType.DMA,
        pltpu.VMEM_SHARED((NP, F), jnp.float32),
    ],
)


BN = 256
GRID = NP // BN


def _mmu_body(x_ref, w_ref, out_ref):
    out_ref[...] = jnp.dot(x_ref[...], w_ref[...],
                           preferred_element_type=jnp.float32)


_mmu = pl.pallas_call(
    _mmu_body,
    grid=(GRID,),
    in_specs=[
        pl.BlockSpec((BN, F), lambda i: (i, 0)),
        pl.BlockSpec((F, F), lambda i: (0, 0)),
    ],
    out_specs=pl.BlockSpec((BN, F), lambda i: (i, 0)),
    out_shape=jax.ShapeDtypeStruct((NP, F), jnp.float32),
)


def _scale_body(y_ref, degs_ref, out_ref):
    ns = lax.rsqrt(jnp.maximum(degs_ref[...], 1.0))
    out_ref[...] = y_ref[...] * ns


_scale = pl.pallas_call(
    _scale_body,
    grid=(GRID,),
    in_specs=[
        pl.BlockSpec((BN, F), lambda i: (i, 0)),
        pl.BlockSpec((BN, 1), lambda i: (i, 0)),
    ],
    out_specs=pl.BlockSpec((BN, F), lambda i: (i, 0)),
    out_shape=jax.ShapeDtypeStruct((NP, F), jnp.float32),
)


def _mid_body(agg_ref, degd_ref, degs_ref, b1_ref, w2_ref, out_ref):
    i = pl.program_id(0)
    a = agg_ref[0] + agg_ref[1]
    nd = lax.rsqrt(jnp.maximum(degd_ref[...], 1.0))
    h = jnp.maximum(a * nd + b1_ref[...], 0.0)
    rid = i * BN + lax.broadcasted_iota(jnp.int32, (BN, 1), 0)
    h = jnp.where(rid < N, h, 0.0)
    ns = lax.rsqrt(jnp.maximum(degs_ref[...], 1.0))
    out_ref[...] = jnp.dot(h * ns, w2_ref[...],
                           preferred_element_type=jnp.float32)


_mid = pl.pallas_call(
    _mid_body,
    grid=(GRID,),
    in_specs=[
        pl.BlockSpec((2, BN, F), lambda i: (0, i, 0)),
        pl.BlockSpec((BN, 1), lambda i: (i, 0)),
        pl.BlockSpec((BN, 1), lambda i: (i, 0)),
        pl.BlockSpec((1, F), lambda i: (0, 0)),
        pl.BlockSpec((F, F), lambda i: (0, 0)),
    ],
    out_specs=pl.BlockSpec((BN, F), lambda i: (i, 0)),
    out_shape=jax.ShapeDtypeStruct((NP, F), jnp.float32),
)


def _fin_body(agg_ref, degd_ref, b2_ref, out_ref):
    a = agg_ref[0] + agg_ref[1]
    nd = lax.rsqrt(jnp.maximum(degd_ref[...], 1.0))
    out_ref[...] = a[:, :64] * nd + b2_ref[...]


_fin = pl.pallas_call(
    _fin_body,
    grid=(GRID,),
    in_specs=[
        pl.BlockSpec((2, BN, F), lambda i: (0, i, 0)),
        pl.BlockSpec((BN, 1), lambda i: (i, 0)),
        pl.BlockSpec((1, 64), lambda i: (0, 0)),
    ],
    out_specs=pl.BlockSpec((BN, 64), lambda i: (i, 0)),
    out_shape=jax.ShapeDtypeStruct((NP, 64), jnp.float32),
)


def kernel(x, edge_index, W1, b1, W2, b2):
    src = edge_index[0]
    dst = edge_index[1]
    pad = N + jnp.arange(EP - E, dtype=jnp.int32) % (NP - N)
    src2d = jnp.concatenate([src, pad]).reshape(NBLK, EB)
    dst2d = jnp.concatenate([dst, pad]).reshape(NBLK, EB)
    xp = jnp.pad(x, ((0, NP - N), (0, 0)))
    w2p = jnp.pad(W2, ((0, 0), (0, F - 64)))

    y1 = _mmu(xp, W1)
    deg_src, deg_dst = _deg_call(src2d, dst2d)
    degs_col = deg_src.reshape(NP, 1)
    degd_col = deg_dst.reshape(NP, 1)

    h1 = _scale(y1, degs_col)
    agg1 = _edge_call(src2d, dst2d, h1)
    h2 = _mid(agg1.reshape(2, NP, F), degd_col, degs_col,
              b1.reshape(1, F), w2p)
    agg2 = _edge_call(src2d, dst2d, h2)
    out = _fin(agg2.reshape(2, NP, F), degd_col, b2.reshape(1, 64))
    return out[:N]


# speedup vs baseline: 1.0949x; 1.0949x over previous
import jax
import jax.numpy as jnp
from jax import lax
from jax.experimental import pallas as pl
from jax.experimental.pallas import tpu as pltpu
from jax.experimental.pallas import tpu_sc as plsc

N = 10000
NP = 10240
E = 320000
EB = 128
NBLK = 2560
EP = NBLK * EB
NC = 2
NS = 16
NBW = NBLK // (NC * NS)
RPS = NP // NS
F = 128

_MESH = plsc.VectorSubcoreMesh(
    core_axis_name="c", subcore_axis_name="s", num_cores=NC, num_subcores=NS)


NBD = NBLK // NS


def _deg_body(src2d, dst2d, deg_src, deg_dst, idx_v, ones_v, zb_v, hist_sh):
    c = lax.axis_index("c")
    sid = lax.axis_index("s")

    one16 = jnp.full((16,), 1.0, jnp.float32)
    zero16 = jnp.zeros((16,), jnp.float32)
    for j in range(EB // 16):
        ones_v[pl.ds(j * 16, 16)] = one16

    @pl.loop(0, RPS // 16)
    def _(j):
        zb_v[pl.ds(j * 16, 16)] = zero16

    pltpu.sync_copy(zb_v, hist_sh.at[pl.ds(sid * RPS, RPS)])

    @pl.when(c == 0)
    def _():
        pltpu.sync_copy(src2d.at[pl.ds(sid * NBD, NBD)], idx_v)

    @pl.when(c == 1)
    def _():
        pltpu.sync_copy(dst2d.at[pl.ds(sid * NBD, NBD)], idx_v)

    plsc.subcore_barrier()

    @pl.loop(0, NBD)
    def _(b):
        pltpu.sync_copy(ones_v, hist_sh.at[idx_v.at[b]], add=True)

    plsc.subcore_barrier()

    pltpu.sync_copy(hist_sh.at[pl.ds(sid * RPS, RPS)], zb_v)

    @pl.when(c == 0)
    def _():
        pltpu.sync_copy(zb_v, deg_src.at[pl.ds(sid * RPS, RPS)])

    @pl.when(c == 1)
    def _():
        pltpu.sync_copy(zb_v, deg_dst.at[pl.ds(sid * RPS, RPS)])


_deg_call = pl.kernel(
    _deg_body,
    out_type=(jax.ShapeDtypeStruct((NP,), jnp.float32),
              jax.ShapeDtypeStruct((NP,), jnp.float32)),
    mesh=_MESH,
    scratch_types=[
        pltpu.VMEM((NBD, EB), jnp.int32),
        pltpu.VMEM((EB,), jnp.float32),
        pltpu.VMEM((RPS,), jnp.float32),
        pltpu.VMEM_SHARED((NP,), jnp.float32),
    ],
)


CHK = 40
NCH = NBW // CHK


def _edge_body(src2d, dst2d, h_hbm, out_hbm,
               srcb, dstb, rows0, rows1, sem0, sem1, agg_sh):
    D = rows0.shape[1]
    c = lax.axis_index("c")
    sid = lax.axis_index("s")
    base = pl.multiple_of(c * NP, NP)
    bb = (c * NS + sid) * NBW

    z16 = jnp.zeros((16,), jnp.float32)

    @pl.loop(0, EB)
    def _(r):
        for j in range(D // 16):
            rows0[r, pl.ds(j * 16, 16)] = z16

    for k in range(RPS // EB):
        pltpu.sync_copy(rows0, agg_sh.at[pl.ds(sid * RPS + k * EB, EB)])
    plsc.subcore_barrier()

    def gstart(b, buf, sem):
        pltpu.async_copy(h_hbm.at[srcb.at[b]], buf, sem)

    def gwait(b, buf, sem):
        pltpu.make_async_copy(h_hbm.at[srcb.at[b]], buf, sem).wait()

    def scat(b, buf):
        pltpu.sync_copy(buf, agg_sh.at[dstb.at[b]], add=True)

    @pl.loop(0, NCH)
    def _(ch):
        pltpu.sync_copy(src2d.at[pl.ds(bb + ch * CHK, CHK)], srcb)
        pltpu.sync_copy(dst2d.at[pl.ds(bb + ch * CHK, CHK)], dstb)
        gstart(0, rows0, sem0)

        @pl.loop(0, CHK // 2)
        def _(i):
            b0 = i * 2
            gstart(b0 + 1, rows1, sem1)
            gwait(b0, rows0, sem0)
            scat(b0, rows0)
            b1 = b0 + 1

            @pl.when(b1 + 1 < CHK)
            def _():
                gstart(b1 + 1, rows0, sem0)

            gwait(b1, rows1, sem1)
            scat(b1, rows1)

    plsc.subcore_barrier()

    for k in range(RPS // EB):
        r = sid * RPS + k * EB
        pltpu.sync_copy(agg_sh.at[pl.ds(r, EB)], rows0)
        pltpu.sync_copy(rows0, out_hbm.at[pl.ds(base + r, EB)])


def _make_edge_call(D, untiled):
    params = (pltpu.CompilerParams(use_tc_tiling_on_sc=False)
              if untiled else None)
    return pl.kernel(
        _edge_body,
        out_type=jax.ShapeDtypeStruct((2 * NP, D), jnp.float32),
        mesh=_MESH,
        compiler_params=params,
        scratch_types=[
            pltpu.VMEM((CHK, EB), jnp.int32),
            pltpu.VMEM((CHK, EB), jnp.int32),
            pltpu.VMEM((EB, D), jnp.float32),
            pltpu.VMEM((EB, D), jnp.float32),
            pltpu.SemaphoreType.DMA,
            pltpu.SemaphoreType.DMA,
            pltpu.VMEM_SHARED((NP, D), jnp.float32),
        ],
    )


_edge_call = _make_edge_call(F, untiled=False)
_edge_call64 = _make_edge_call(64, untiled=True)


BN = 256
GRID = NP // BN


def _mm1_body(x_ref, degs_ref, w_ref, out_ref):
    ns = lax.rsqrt(jnp.maximum(degs_ref[...], 1.0))
    out_ref[...] = jnp.dot(x_ref[...] * ns, w_ref[...],
                           preferred_element_type=jnp.float32)


_mm1 = pl.pallas_call(
    _mm1_body,
    grid=(GRID,),
    in_specs=[
        pl.BlockSpec((BN, F), lambda i: (i, 0)),
        pl.BlockSpec((BN, 1), lambda i: (i, 0)),
        pl.BlockSpec((F, F), lambda i: (0, 0)),
    ],
    out_specs=pl.BlockSpec((BN, F), lambda i: (i, 0)),
    out_shape=jax.ShapeDtypeStruct((NP, F), jnp.float32),
)


def _mid_body(agg_ref, degd_ref, degs_ref, b1_ref, w2_ref, out_ref):
    i = pl.program_id(0)
    a = agg_ref[0] + agg_ref[1]
    nd = lax.rsqrt(jnp.maximum(degd_ref[...], 1.0))
    h = jnp.maximum(a * nd + b1_ref[...], 0.0)
    rid = i * BN + lax.broadcasted_iota(jnp.int32, (BN, 1), 0)
    h = jnp.where(rid < N, h, 0.0)
    ns = lax.rsqrt(jnp.maximum(degs_ref[...], 1.0))
    out_ref[...] = jnp.dot(h * ns, w2_ref[...],
                           preferred_element_type=jnp.float32)


_mid = pl.pallas_call(
    _mid_body,
    grid=(GRID,),
    in_specs=[
        pl.BlockSpec((2, BN, F), lambda i: (0, i, 0)),
        pl.BlockSpec((BN, 1), lambda i: (i, 0)),
        pl.BlockSpec((BN, 1), lambda i: (i, 0)),
        pl.BlockSpec((1, F), lambda i: (0, 0)),
        pl.BlockSpec((F, 64), lambda i: (0, 0)),
    ],
    out_specs=pl.BlockSpec((BN, 64), lambda i: (i, 0)),
    out_shape=jax.ShapeDtypeStruct((NP, 64), jnp.float32),
)


def _fin_body(agg_ref, degd_ref, b2_ref, out_ref):
    a = agg_ref[0] + agg_ref[1]
    nd = lax.rsqrt(jnp.maximum(degd_ref[...], 1.0))
    out_ref[...] = a * nd + b2_ref[...]


_fin = pl.pallas_call(
    _fin_body,
    grid=(GRID,),
    in_specs=[
        pl.BlockSpec((2, BN, 64), lambda i: (0, i, 0)),
        pl.BlockSpec((BN, 1), lambda i: (i, 0)),
        pl.BlockSpec((1, 64), lambda i: (0, 0)),
    ],
    out_specs=pl.BlockSpec((BN, 64), lambda i: (i, 0)),
    out_shape=jax.ShapeDtypeStruct((NP, 64), jnp.float32),
)


def kernel(x, edge_index, W1, b1, W2, b2):
    src = edge_index[0]
    dst = edge_index[1]
    pad = N + jnp.arange(EP - E, dtype=jnp.int32) % (NP - N)
    src2d = jnp.concatenate([src, pad]).reshape(NBLK, EB)
    dst2d = jnp.concatenate([dst, pad]).reshape(NBLK, EB)
    xp = jnp.pad(x, ((0, NP - N), (0, 0)))

    deg_src, deg_dst = _deg_call(src2d, dst2d)
    degs_col = deg_src.reshape(NP, 1)
    degd_col = deg_dst.reshape(NP, 1)

    h1 = _mm1(xp, degs_col, W1)
    agg1 = _edge_call(src2d, dst2d, h1)
    h2 = _mid(agg1.reshape(2, NP, F), degd_col, degs_col,
              b1.reshape(1, F), W2)
    agg2 = _edge_call64(src2d, dst2d, h2)
    out = _fin(agg2.reshape(2, NP, 64), degd_col, b2.reshape(1, 64))
    return out[:N]


# speedup vs baseline: 1.0950x; 1.0000x over previous
import jax
import jax.numpy as jnp
from jax import lax
from jax.experimental import pallas as pl
from jax.experimental.pallas import tpu as pltpu
from jax.experimental.pallas import tpu_sc as plsc

N = 10000
NP = 10240
E = 320000
EB = 128
NBLK = 2560
EP = NBLK * EB
NC = 2
NS = 16
NBW = NBLK // (NC * NS)
RPS = NP // NS
F = 128

_MESH = plsc.VectorSubcoreMesh(
    core_axis_name="c", subcore_axis_name="s", num_cores=NC, num_subcores=NS)


NBD = NBLK // NS


def _deg_body(src2d, dst2d, deg_src, deg_dst, idx_v, ones_v, zb_v, hist_sh):
    c = lax.axis_index("c")
    sid = lax.axis_index("s")

    one16 = jnp.full((16,), 1.0, jnp.float32)
    zero16 = jnp.zeros((16,), jnp.float32)
    for j in range(EB // 16):
        ones_v[pl.ds(j * 16, 16)] = one16

    @pl.loop(0, RPS // 16)
    def _(j):
        zb_v[pl.ds(j * 16, 16)] = zero16

    pltpu.sync_copy(zb_v, hist_sh.at[pl.ds(sid * RPS, RPS)])

    @pl.when(c == 0)
    def _():
        pltpu.sync_copy(src2d.at[pl.ds(sid * NBD, NBD)], idx_v)

    @pl.when(c == 1)
    def _():
        pltpu.sync_copy(dst2d.at[pl.ds(sid * NBD, NBD)], idx_v)

    plsc.subcore_barrier()

    @pl.loop(0, NBD)
    def _(b):
        pltpu.sync_copy(ones_v, hist_sh.at[idx_v.at[b]], add=True)

    plsc.subcore_barrier()

    pltpu.sync_copy(hist_sh.at[pl.ds(sid * RPS, RPS)], zb_v)

    @pl.when(c == 0)
    def _():
        pltpu.sync_copy(zb_v, deg_src.at[pl.ds(sid * RPS, RPS)])

    @pl.when(c == 1)
    def _():
        pltpu.sync_copy(zb_v, deg_dst.at[pl.ds(sid * RPS, RPS)])


_deg_call = pl.kernel(
    _deg_body,
    out_type=(jax.ShapeDtypeStruct((NP,), jnp.float32),
              jax.ShapeDtypeStruct((NP,), jnp.float32)),
    mesh=_MESH,
    compiler_params=pltpu.CompilerParams(use_tc_tiling_on_sc=False),
    scratch_types=[
        pltpu.VMEM((NBD, EB), jnp.int32),
        pltpu.VMEM((EB,), jnp.float32),
        pltpu.VMEM((RPS,), jnp.float32),
        pltpu.VMEM_SHARED((NP,), jnp.float32),
    ],
)


CHK = 40
NCH = NBW // CHK


def _edge_body(src2d, dst2d, h_hbm, out_hbm,
               srcb, dstb, rows0, rows1, sem0, sem1, agg_sh):
    D = rows0.shape[1]
    c = lax.axis_index("c")
    sid = lax.axis_index("s")
    base = pl.multiple_of(c * NP, NP)
    bb = (c * NS + sid) * NBW

    z16 = jnp.zeros((16,), jnp.float32)

    @pl.loop(0, EB)
    def _(r):
        for j in range(D // 16):
            rows0[r, pl.ds(j * 16, 16)] = z16

    for k in range(RPS // EB):
        pltpu.sync_copy(rows0, agg_sh.at[pl.ds(sid * RPS + k * EB, EB)])
    plsc.subcore_barrier()

    def gstart(b, buf, sem):
        pltpu.async_copy(h_hbm.at[srcb.at[b]], buf, sem)

    def gwait(b, buf, sem):
        pltpu.make_async_copy(h_hbm.at[srcb.at[b]], buf, sem).wait()

    def scat(b, buf):
        pltpu.sync_copy(buf, agg_sh.at[dstb.at[b]], add=True)

    @pl.loop(0, NCH)
    def _(ch):
        pltpu.sync_copy(src2d.at[pl.ds(bb + ch * CHK, CHK)], srcb)
        pltpu.sync_copy(dst2d.at[pl.ds(bb + ch * CHK, CHK)], dstb)
        gstart(0, rows0, sem0)

        @pl.loop(0, CHK // 2)
        def _(i):
            b0 = i * 2
            gstart(b0 + 1, rows1, sem1)
            gwait(b0, rows0, sem0)
            scat(b0, rows0)
            b1 = b0 + 1

            @pl.when(b1 + 1 < CHK)
            def _():
                gstart(b1 + 1, rows0, sem0)

            gwait(b1, rows1, sem1)
            scat(b1, rows1)

    plsc.subcore_barrier()

    for k in range(RPS // EB):
        r = sid * RPS + k * EB
        pltpu.sync_copy(agg_sh.at[pl.ds(r, EB)], rows0)
        pltpu.sync_copy(rows0, out_hbm.at[pl.ds(base + r, EB)])


def _make_edge_call(D, untiled):
    params = (pltpu.CompilerParams(use_tc_tiling_on_sc=False)
              if untiled else None)
    return pl.kernel(
        _edge_body,
        out_type=jax.ShapeDtypeStruct((2 * NP, D), jnp.float32),
        mesh=_MESH,
        compiler_params=params,
        scratch_types=[
            pltpu.VMEM((CHK, EB), jnp.int32),
            pltpu.VMEM((CHK, EB), jnp.int32),
            pltpu.VMEM((EB, D), jnp.float32),
            pltpu.VMEM((EB, D), jnp.float32),
            pltpu.SemaphoreType.DMA,
            pltpu.SemaphoreType.DMA,
            pltpu.VMEM_SHARED((NP, D), jnp.float32),
        ],
    )


_edge_call = _make_edge_call(F, untiled=True)
_edge_call64 = _make_edge_call(64, untiled=True)


BN = 256
GRID = NP // BN


def _mm1_body(x_ref, degs_ref, w_ref, out_ref):
    ns = lax.rsqrt(jnp.maximum(degs_ref[...], 1.0))
    out_ref[...] = jnp.dot(x_ref[...] * ns, w_ref[...],
                           preferred_element_type=jnp.float32)


_mm1 = pl.pallas_call(
    _mm1_body,
    grid=(GRID,),
    in_specs=[
        pl.BlockSpec((BN, F), lambda i: (i, 0)),
        pl.BlockSpec((BN, 1), lambda i: (i, 0)),
        pl.BlockSpec((F, F), lambda i: (0, 0)),
    ],
    out_specs=pl.BlockSpec((BN, F), lambda i: (i, 0)),
    out_shape=jax.ShapeDtypeStruct((NP, F), jnp.float32),
)


def _mid_body(agg_ref, degd_ref, degs_ref, b1_ref, w2_ref, out_ref):
    i = pl.program_id(0)
    a = agg_ref[0] + agg_ref[1]
    nd = lax.rsqrt(jnp.maximum(degd_ref[...], 1.0))
    h = jnp.maximum(a * nd + b1_ref[...], 0.0)
    rid = i * BN + lax.broadcasted_iota(jnp.int32, (BN, 1), 0)
    h = jnp.where(rid < N, h, 0.0)
    ns = lax.rsqrt(jnp.maximum(degs_ref[...], 1.0))
    out_ref[...] = jnp.dot(h * ns, w2_ref[...],
                           preferred_element_type=jnp.float32)


_mid = pl.pallas_call(
    _mid_body,
    grid=(GRID,),
    in_specs=[
        pl.BlockSpec((2, BN, F), lambda i: (0, i, 0)),
        pl.BlockSpec((BN, 1), lambda i: (i, 0)),
        pl.BlockSpec((BN, 1), lambda i: (i, 0)),
        pl.BlockSpec((1, F), lambda i: (0, 0)),
        pl.BlockSpec((F, 64), lambda i: (0, 0)),
    ],
    out_specs=pl.BlockSpec((BN, 64), lambda i: (i, 0)),
    out_shape=jax.ShapeDtypeStruct((NP, 64), jnp.float32),
)


def _fin_body(agg_ref, degd_ref, b2_ref, out_ref):
    a = agg_ref[0] + agg_ref[1]
    nd = lax.rsqrt(jnp.maximum(degd_ref[...], 1.0))
    out_ref[...] = a * nd + b2_ref[...]


_fin = pl.pallas_call(
    _fin_body,
    grid=(GRID,),
    in_specs=[
        pl.BlockSpec((2, BN, 64), lambda i: (0, i, 0)),
        pl.BlockSpec((BN, 1), lambda i: (i, 0)),
        pl.BlockSpec((1, 64), lambda i: (0, 0)),
    ],
    out_specs=pl.BlockSpec((BN, 64), lambda i: (i, 0)),
    out_shape=jax.ShapeDtypeStruct((NP, 64), jnp.float32),
)


def kernel(x, edge_index, W1, b1, W2, b2):
    src = edge_index[0]
    dst = edge_index[1]
    pad = N + jnp.arange(EP - E, dtype=jnp.int32) % (NP - N)
    src2d = jnp.concatenate([src, pad]).reshape(NBLK, EB)
    dst2d = jnp.concatenate([dst, pad]).reshape(NBLK, EB)
    xp = jnp.pad(x, ((0, NP - N), (0, 0)))

    deg_src, deg_dst = _deg_call(src2d, dst2d)
    degs_col = deg_src.reshape(NP, 1)
    degd_col = deg_dst.reshape(NP, 1)

    h1 = _mm1(xp, degs_col, W1)
    agg1 = _edge_call(src2d, dst2d, h1)
    h2 = _mid(agg1.reshape(2, NP, F), degd_col, degs_col,
              b1.reshape(1, F), W2)
    agg2 = _edge_call64(src2d, dst2d, h2)
    out = _fin(agg2.reshape(2, NP, 64), degd_col, b2.reshape(1, 64))
    return out[:N]
